# baseline (speedup 1.0000x reference)
import functools

import jax
import jax.numpy as jnp
from jax import lax
from jax.experimental import pallas as pl
from jax.experimental.pallas import tpu as pltpu
from jax.experimental.pallas import tpu_sc as plsc

B = 16384
D = 64
H = B // 2
C = 1000
C1 = 1024
TW = 80
MARGIN = 0.3
EPS = 1e-12

NC = 2
NS = 16
NW = NC * NS
RPW = B // NW
LPW = RPW // 2
NCHUNK = RPW // 64

_SC_MESH = plsc.VectorSubcoreMesh(core_axis_name="c", subcore_axis_name="s")
_SC_PARAMS = pltpu.CompilerParams(needs_layout_passes=False,
                                  use_tc_tiling_on_sc=False)


def _norm_body(x_ref, o_ref):
    x = x_ref[...]
    s = jnp.sum(x * x, axis=1, keepdims=True)
    xn = x * (1.0 / jnp.maximum(jnp.sqrt(s), EPS))
    o_ref[...] = jnp.concatenate([xn[0:H, :], xn[H:B, :]], axis=1)


def _segsum_body(emb_hbm, lab_hbm, z80_hbm, st_hbm,
                 en2_v, sc_v, lab_v, ssum, sem):
    cid = lax.axis_index("c")
    sid = lax.axis_index("s")
    wid = cid * NS + sid
    rows = C1 // NS

    hs = [pltpu.async_copy(z80_hbm.at[pl.ds(sid * rows, rows)],
                           ssum.at[pl.ds(sid * rows, rows)], sem),
          pltpu.async_copy(emb_hbm.at[pl.ds(wid * LPW, LPW)], en2_v, sem),
          pltpu.async_copy(lab_hbm.at[wid], lab_v, sem)]
    for h in hs:
        h.wait()

    onehot = (lax.broadcasted_iota(jnp.int32, (16,), 0) == 0).astype(jnp.float32)

    @pl.loop(0, LPW // 16)
    def _(g):
        for k in range(16):
            i = g * 16 + k
            for m in range(4):
                sc_v[i, pl.ds(m * 16, 16)] = en2_v[i, pl.ds(m * 16, 16)]
                sc_v[LPW + i, pl.ds(m * 16, 16)] = en2_v[i, pl.ds(64 + m * 16, 16)]
            sc_v[i, pl.ds(D, 16)] = onehot
            sc_v[LPW + i, pl.ds(D, 16)] = onehot

    plsc.subcore_barrier()

    hs = [pltpu.async_copy(sc_v.at[pl.ds(j * 64, 64)],
                           ssum.at[lab_v.at[j, 0]], sem, add=True)
          for j in range(NCHUNK)]
    for h in hs:
        h.wait()
    plsc.subcore_barrier()

    pltpu.sync_copy(ssum.at[pl.ds(sid * rows, rows)],
                    st_hbm.at[cid, pl.ds(sid * rows, rows)])


_segsum_call = functools.partial(
    pl.kernel, _segsum_body,
    out_type=jax.ShapeDtypeStruct((NC, C1, TW), jnp.float32),
    mesh=_SC_MESH,
    scratch_types=[
        pltpu.VMEM((LPW, 128), jnp.float32),
        pltpu.VMEM((RPW, TW), jnp.float32),
        pltpu.VMEM((NCHUNK, 1, 64), jnp.int32),
        pltpu.VMEM_SHARED((C1, TW), jnp.float32),
        pltpu.SemaphoreType.DMA,
    ],
    compiler_params=_SC_PARAMS,
)()


def _centroid_body(st_ref, t_ref):
    sums = st_ref[0, :, 0:D] + st_ref[1, :, 0:D]
    counts = st_ref[0, :, D:D + 1] + st_ref[1, :, D:D + 1]
    safe = jnp.maximum(counts, 1.0)
    cen = sums / safe
    cn = jnp.maximum(jnp.sqrt(jnp.sum(cen * cen, axis=1, keepdims=True)), EPS)
    cen = cen / cn

    cen2 = cen * cen
    sq_col = jnp.sum(cen2, axis=1, keepdims=True)
    ones_row = jnp.ones((1, D), jnp.float32)
    sq_row = lax.dot_general(ones_row, cen2, (((1,), (1,)), ((), ())),
                             preferred_element_type=jnp.float32)
    g = lax.dot_general(cen, cen, (((1,), (1,)), ((), ())),
                        preferred_element_type=jnp.float32)
    d2 = jnp.maximum(sq_col + sq_row - 2.0 * g, 0.0)
    dist = jnp.sqrt(d2)
    row_i = lax.broadcasted_iota(jnp.int32, (C1, C1), 0)
    col_i = lax.broadcasted_iota(jnp.int32, (C1, C1), 1)
    dist = jnp.where((row_i == col_i) | (col_i >= C), jnp.inf, dist)
    minv = jnp.min(dist, axis=1, keepdims=True)
    nearest = jnp.min(jnp.where(dist == minv, col_i, jnp.int32(2 ** 30)),
                      axis=1, keepdims=True)

    oh_n = (col_i == nearest).astype(jnp.float32)
    c_neg = lax.dot_general(oh_n, cen, (((1,), (0,)), ((), ())),
                            preferred_element_type=jnp.float32)

    present = (counts > 0.0).astype(jnp.float32)
    den = jnp.maximum(jnp.sum(present, axis=0, keepdims=True), 1.0)
    s = present / (safe * den)
    lane = lax.broadcasted_iota(jnp.int32, (C1, 128 - D), 1)
    scale_cols = s * (lane == 0).astype(jnp.float32)
    t_ref[...] = jnp.concatenate([c_neg - cen, scale_cols], axis=1)


def _vals_body(en_hbm, lab_hbm, t_hbm, out_hbm,
               en2_v, t_rows, lab_v, acc_v, sem, sem_idx):
    cid = lax.axis_index("c")
    sid = lax.axis_index("s")
    wid = cid * NS + sid

    h1 = pltpu.async_copy(en_hbm.at[pl.ds(wid * LPW, LPW)], en2_v, sem)
    h2 = pltpu.async_copy(lab_hbm.at[wid], lab_v, sem_idx)
    h2.wait()

    hs = [pltpu.async_copy(t_hbm.at[lab_v.at[j, 0]],
                           t_rows.at[pl.ds(j * 64, 64)], sem)
          for j in range(NCHUNK)]
    h1.wait()
    for h in hs:
        h.wait()

    def body(off, g, acc):
        for k in range(16):
            i = g * 16 + k
            a = off + i
            p = en2_v[i, pl.ds(off // 4, 16)] * t_rows[a, pl.ds(0, 16)]
            for m in range(1, 4):
                p = p + (en2_v[i, pl.ds(off // 4 + m * 16, 16)]
                         * t_rows[a, pl.ds(m * 16, 16)])
            t = jnp.sum(p)
            v = jnp.maximum(t + MARGIN, 0.0)
            acc = acc + v * t_rows[a, pl.ds(D, 16)]
        return acc

    acc = lax.fori_loop(0, LPW // 16, functools.partial(body, 0),
                        jnp.zeros((16,), jnp.float32))
    acc = lax.fori_loop(0, LPW // 16, functools.partial(body, LPW), acc)
    acc_v[...] = acc
    pltpu.sync_copy(acc_v, out_hbm.at[cid, sid])


_vals_call = functools.partial(
    pl.kernel, _vals_body,
    out_type=jax.ShapeDtypeStruct((NC, NS, 16), jnp.float32),
    mesh=_SC_MESH,
    scratch_types=[
        pltpu.VMEM((LPW, 128), jnp.float32),
        pltpu.VMEM((RPW, 128), jnp.float32),
        pltpu.VMEM((NCHUNK, 1, 64), jnp.int32),
        pltpu.VMEM((16,), jnp.float32),
        pltpu.SemaphoreType.DMA,
        pltpu.SemaphoreType.DMA,
    ],
    compiler_params=_SC_PARAMS,
)()


def _finalize_body(part_ref, out_ref):
    p = part_ref[0, :, :] + part_ref[1, :, :]
    num = jnp.sum(p, axis=0, keepdims=True)
    out_ref[...] = jnp.sum(num, axis=1, keepdims=True)


def kernel(embeddings, labels):
    en2 = pl.pallas_call(
        _norm_body,
        out_shape=jax.ShapeDtypeStruct((H, 128), jnp.float32),
    )(embeddings)

    lab_chunks = (labels.reshape(2, NW, NCHUNK // 2, 64)
                  .transpose(1, 0, 2, 3).reshape(NW, NCHUNK, 1, 64))
    z80 = jnp.zeros((C1, TW), jnp.float32)
    st = _segsum_call(en2, lab_chunks, z80)

    t = pl.pallas_call(
        _centroid_body,
        out_shape=jax.ShapeDtypeStruct((C1, 128), jnp.float32),
    )(st)

    part = _vals_call(en2, lab_chunks, t)

    out = pl.pallas_call(
        _finalize_body,
        out_shape=jax.ShapeDtypeStruct((1, 1), jnp.float32),
    )(part)
    return out[0, 0]


# speedup vs baseline: 1.2378x; 1.2378x over previous
import functools

import jax
import jax.numpy as jnp
from jax import lax
from jax.experimental import pallas as pl
from jax.experimental.pallas import tpu as pltpu
from jax.experimental.pallas import tpu_sc as plsc

B = 16384
D = 64
H = B // 2
C = 1000
C1 = 1024
TW = 80
MARGIN = 0.3
EPS = 1e-12

NC = 2
NS = 16
NW = NC * NS
RPW = B // NW
LPW = RPW // 2
NCHUNK = RPW // 64

_SC_MESH = plsc.VectorSubcoreMesh(core_axis_name="c", subcore_axis_name="s")
_SC_PARAMS = pltpu.CompilerParams(needs_layout_passes=False,
                                  use_tc_tiling_on_sc=False)


def _norm_body(xt_ref, o_ref):
    xt = xt_ref[...]
    ss = jnp.sum(xt * xt, axis=0, keepdims=True)
    xnt = xt * (1.0 / jnp.maximum(jnp.sqrt(ss), EPS))
    o_ref[...] = jnp.concatenate([xnt[:, 0:H], xnt[:, H:B]], axis=0).T


def _segsum_body(emb_hbm, lab_hbm, z80_hbm, st_hbm,
                 en2_v, sc_v, lab_v, ssum, sem):
    cid = lax.axis_index("c")
    sid = lax.axis_index("s")
    wid = cid * NS + sid
    rows = C1 // NS

    hs = [pltpu.async_copy(z80_hbm.at[pl.ds(sid * rows, rows)],
                           ssum.at[pl.ds(sid * rows, rows)], sem),
          pltpu.async_copy(emb_hbm.at[pl.ds(wid * LPW, LPW)], en2_v, sem),
          pltpu.async_copy(lab_hbm.at[pl.ds(wid * LPW, LPW)],
                           lab_v.at[0], sem),
          pltpu.async_copy(lab_hbm.at[pl.ds(H + wid * LPW, LPW)],
                           lab_v.at[1], sem)]
    for h in hs:
        h.wait()

    onehot = (lax.broadcasted_iota(jnp.int32, (16,), 0) == 0).astype(jnp.float32)

    @pl.loop(0, LPW // 16)
    def _(g):
        for k in range(16):
            i = g * 16 + k
            for m in range(4):
                sc_v[i, pl.ds(m * 16, 16)] = en2_v[i, pl.ds(m * 16, 16)]
                sc_v[LPW + i, pl.ds(m * 16, 16)] = en2_v[i, pl.ds(64 + m * 16, 16)]
            sc_v[i, pl.ds(D, 16)] = onehot
            sc_v[LPW + i, pl.ds(D, 16)] = onehot

    plsc.subcore_barrier()

    hs = [pltpu.async_copy(sc_v.at[pl.ds(j * 64, 64)],
                           ssum.at[lab_v.at[j // 4, pl.ds((j % 4) * 64, 64)]],
                           sem, add=True)
          for j in range(NCHUNK)]
    for h in hs:
        h.wait()
    plsc.subcore_barrier()

    pltpu.sync_copy(ssum.at[pl.ds(sid * rows, rows)],
                    st_hbm.at[cid, pl.ds(sid * rows, rows)])


_segsum_call = functools.partial(
    pl.kernel, _segsum_body,
    out_type=jax.ShapeDtypeStruct((NC, C1, TW), jnp.float32),
    mesh=_SC_MESH,
    scratch_types=[
        pltpu.VMEM((LPW, 128), jnp.float32),
        pltpu.VMEM((RPW, TW), jnp.float32),
        pltpu.VMEM((2, LPW), jnp.int32),
        pltpu.VMEM_SHARED((C1, TW), jnp.float32),
        pltpu.SemaphoreType.DMA,
    ],
    compiler_params=_SC_PARAMS,
)()


def _centroid_body(st_ref, t_ref):
    sums = st_ref[0, :, 0:D] + st_ref[1, :, 0:D]
    counts = st_ref[0, :, D:D + 1] + st_ref[1, :, D:D + 1]
    safe = jnp.maximum(counts, 1.0)
    cen = sums / safe
    cn = jnp.maximum(jnp.sqrt(jnp.sum(cen * cen, axis=1, keepdims=True)), EPS)
    cen = cen / cn

    cen2 = cen * cen
    sq_col = jnp.sum(cen2, axis=1, keepdims=True)
    ones_row = jnp.ones((1, D), jnp.float32)
    sq_row = lax.dot_general(ones_row, cen2, (((1,), (1,)), ((), ())),
                             preferred_element_type=jnp.float32)
    g = lax.dot_general(cen, cen, (((1,), (1,)), ((), ())),
                        preferred_element_type=jnp.float32)
    d2 = jnp.maximum(sq_col + sq_row - 2.0 * g, 0.0)
    dist = jnp.sqrt(d2)
    row_i = lax.broadcasted_iota(jnp.int32, (C1, C1), 0)
    col_i = lax.broadcasted_iota(jnp.int32, (C1, C1), 1)
    dist = jnp.where((row_i == col_i) | (col_i >= C), jnp.inf, dist)
    minv = jnp.min(dist, axis=1, keepdims=True)
    nearest = jnp.min(jnp.where(dist == minv, col_i, jnp.int32(2 ** 30)),
                      axis=1, keepdims=True)

    oh_n = (col_i == nearest).astype(jnp.float32)
    c_neg = lax.dot_general(oh_n, cen, (((1,), (0,)), ((), ())),
                            preferred_element_type=jnp.float32)

    present = (counts > 0.0).astype(jnp.float32)
    den = jnp.maximum(jnp.sum(present, axis=0, keepdims=True), 1.0)
    s = present / (safe * den)
    lane = lax.broadcasted_iota(jnp.int32, (C1, 128 - D), 1)
    scale_cols = s * (lane == 0).astype(jnp.float32)
    t_ref[...] = jnp.concatenate([c_neg - cen, scale_cols], axis=1)


def _vals_body(en_hbm, lab_hbm, t_hbm, out_hbm,
               en2_v, t_rows, lab_v, acc_v, sem, sem_idx):
    cid = lax.axis_index("c")
    sid = lax.axis_index("s")
    wid = cid * NS + sid

    h1 = pltpu.async_copy(en_hbm.at[pl.ds(wid * LPW, LPW)], en2_v, sem)
    h2 = pltpu.async_copy(lab_hbm.at[pl.ds(wid * LPW, LPW)],
                          lab_v.at[0], sem_idx)
    h3 = pltpu.async_copy(lab_hbm.at[pl.ds(H + wid * LPW, LPW)],
                          lab_v.at[1], sem_idx)
    h2.wait()
    h3.wait()

    hs = [pltpu.async_copy(t_hbm.at[lab_v.at[j // 4, pl.ds((j % 4) * 64, 64)]],
                           t_rows.at[pl.ds(j * 64, 64)], sem)
          for j in range(NCHUNK)]
    h1.wait()
    for h in hs:
        h.wait()

    def body(off, g, acc):
        for k in range(16):
            i = g * 16 + k
            a = off + i
            p = en2_v[i, pl.ds(off // 4, 16)] * t_rows[a, pl.ds(0, 16)]
            for m in range(1, 4):
                p = p + (en2_v[i, pl.ds(off // 4 + m * 16, 16)]
                         * t_rows[a, pl.ds(m * 16, 16)])
            t = jnp.sum(p)
            v = jnp.maximum(t + MARGIN, 0.0)
            acc = acc + v * t_rows[a, pl.ds(D, 16)]
        return acc

    acc = lax.fori_loop(0, LPW // 16, functools.partial(body, 0),
                        jnp.zeros((16,), jnp.float32))
    acc = lax.fori_loop(0, LPW // 16, functools.partial(body, LPW), acc)
    acc_v[...] = acc
    pltpu.sync_copy(acc_v, out_hbm.at[cid, sid])


_vals_call = functools.partial(
    pl.kernel, _vals_body,
    out_type=jax.ShapeDtypeStruct((NC, NS, 16), jnp.float32),
    mesh=_SC_MESH,
    scratch_types=[
        pltpu.VMEM((LPW, 128), jnp.float32),
        pltpu.VMEM((RPW, 128), jnp.float32),
        pltpu.VMEM((2, LPW), jnp.int32),
        pltpu.VMEM((16,), jnp.float32),
        pltpu.SemaphoreType.DMA,
        pltpu.SemaphoreType.DMA,
    ],
    compiler_params=_SC_PARAMS,
)()


def _finalize_body(part_ref, out_ref):
    p = part_ref[0, :, :] + part_ref[1, :, :]
    num = jnp.sum(p, axis=0, keepdims=True)
    out_ref[...] = jnp.sum(num, axis=1, keepdims=True)


def kernel(embeddings, labels):
    en2 = pl.pallas_call(
        _norm_body,
        out_shape=jax.ShapeDtypeStruct((H, 128), jnp.float32),
    )(embeddings.T)

    z80 = jnp.zeros((C1, TW), jnp.float32)
    st = _segsum_call(en2, labels, z80)

    t = pl.pallas_call(
        _centroid_body,
        out_shape=jax.ShapeDtypeStruct((C1, 128), jnp.float32),
    )(st)

    part = _vals_call(en2, labels, t)

    out = pl.pallas_call(
        _finalize_body,
        out_shape=jax.ShapeDtypeStruct((1, 1), jnp.float32),
    )(part)
    return out[0, 0]
